# Initial kernel scaffold; baseline (speedup 1.0000x reference)
import jax
import jax.numpy as jnp
from jax.experimental import pallas as pl


def kernel(zone_features, current_zone_ids, time_step, W_spatial, b_spatial, W_feat, b_feat, W_time, b_time, table):
    raise NotImplementedError("write your pallas kernel here")


# speedup vs baseline: 3.1558x; 3.1558x over previous
import functools

import jax
import jax.numpy as jnp
from jax import lax
from jax.experimental import pallas as pl
from jax.experimental.pallas import tpu as pltpu
from jax.experimental.pallas import tpu_sc as plsc

B, P, Z, F = 1024, 50, 100, 11
ZED, TED = 64, 32
N = B * P
NC, NS = 2, 16
NW = NC * NS
BPW = N // NW
NG = BPW // 16
GCH = 128
NGCH = 2 * BPW // GCH
ZPAD = 128
BB = 128
GRID = B // BB


def _sc_gather(zfg, gidx, shifts):
    mesh = plsc.VectorSubcoreMesh(core_axis_name="c", subcore_axis_name="s")

    @functools.partial(
        pl.kernel,
        mesh=mesh,
        out_type=jax.ShapeDtypeStruct((N, F), jnp.float32),
        scratch_types=[
            pltpu.VMEM((NGCH, GCH), jnp.int32),
            pltpu.VMEM((2 * BPW, 16), jnp.float32),
            pltpu.VMEM((BPW,), jnp.int32),
            pltpu.VMEM((BPW, F), jnp.float32),
            pltpu.SemaphoreType.DMA,
        ],
        compiler_params=pltpu.CompilerParams(use_tc_tiling_on_sc=False,
                                             needs_layout_passes=False),
    )
    def k(zf_hbm, gidx_hbm, s_hbm, out_hbm, idx_v, win_v, s_v, rows_v, sem):
        wid = lax.axis_index("s") * NC + lax.axis_index("c")
        base = wid * BPW
        pltpu.sync_copy(gidx_hbm.at[wid], idx_v)
        pltpu.sync_copy(s_hbm.at[wid], s_v)
        copies = [
            pltpu.async_copy(
                zf_hbm.at[idx_v.at[k2]], win_v.at[pl.ds(k2 * GCH, GCH)], sem
            )
            for k2 in range(NGCH)
        ]
        for c in copies:
            c.wait()

        iota = lax.iota(jnp.int32, 16)
        i32x = iota * 32
        zeros = iota * 0

        def group(j, carry):
            nvec = iota + j * 16
            sv = s_v[pl.ds(j * 16, 16)]
            p0 = i32x + sv + j * 512
            for c in range(F):
                p = p0 + c
                v = plsc.load_gather(win_v, [p >> 4, p & 15])
                plsc.store_scatter(rows_v, [nvec, zeros + c], v)
            return carry

        lax.fori_loop(0, NG, group, 0)
        pltpu.sync_copy(rows_v, out_hbm.at[pl.ds(base, BPW)])

    return k(zfg, gidx, shifts)


def _tc_body(cur_ref, ids_ref, t_ref, wc_ref, bc_ref, wt_ref, bt_ref,
             tab_ref, out_ref):
    cur = cur_ref[...]
    ids = ids_ref[...]
    onehot = (ids == lax.broadcasted_iota(jnp.int32, (1, ZPAD), 1)
              ).astype(jnp.float32)
    zone = jnp.dot(cur, wc_ref[...],
                   preferred_element_type=jnp.float32,
                   precision=lax.Precision.HIGHEST)
    zone = zone + jnp.dot(onehot, tab_ref[...],
                          preferred_element_type=jnp.float32,
                          precision=lax.Precision.HIGHEST)
    zone = zone + bc_ref[...]
    te = t_ref[...] * wt_ref[...] + bt_ref[...]
    te_b = jnp.broadcast_to(te[:, None, :], (BB, P, TED)).reshape(BB * P, TED)
    out_ref[...] = jnp.concatenate([zone, te_b], axis=-1)


def _tc_dense(cur, ids_col, time_step, w_comb, b_comb, wt_row, bt_row,
              table_pad):
    return pl.pallas_call(
        _tc_body,
        grid=(GRID,),
        in_specs=[
            pl.BlockSpec((BB * P, F), lambda i: (i, 0)),
            pl.BlockSpec((BB * P, 1), lambda i: (i, 0)),
            pl.BlockSpec((BB, 1), lambda i: (i, 0)),
            pl.BlockSpec((F, ZED), lambda i: (0, 0)),
            pl.BlockSpec((1, ZED), lambda i: (0, 0)),
            pl.BlockSpec((1, TED), lambda i: (0, 0)),
            pl.BlockSpec((1, TED), lambda i: (0, 0)),
            pl.BlockSpec((ZPAD, ZED), lambda i: (0, 0)),
        ],
        out_specs=pl.BlockSpec((BB * P, ZED + TED), lambda i: (i, 0)),
        out_shape=jax.ShapeDtypeStruct((N, ZED + TED), jnp.float32),
    )(cur, ids_col, time_step, w_comb, b_comb, wt_row, bt_row, table_pad)


def kernel(zone_features, current_zone_ids, time_step, W_spatial, b_spatial,
           W_feat, b_feat, W_time, b_time, table):
    ids = current_zone_ids.astype(jnp.int32)
    ids_flat = ids.reshape(N)

    f = jnp.arange(N, dtype=jnp.int32) * Z + ids_flat
    p0 = f * F
    g = p0 >> 4
    gidx = jnp.stack([g, g + 1], axis=-1).reshape(NW, NGCH, GCH)
    shifts = (p0 & 15).reshape(NW, BPW)
    zfg = zone_features.reshape(N * Z * F // 16, 16)

    cur = _sc_gather(zfg, gidx, shifts)

    w_comb = (jnp.zeros((F, ZED), jnp.float32)
              .at[:2, :ZED // 2].set(W_spatial.T)
              .at[2:, ZED // 2:].set(W_feat.T))
    b_comb = jnp.concatenate([b_spatial, b_feat]).reshape(1, ZED)
    wt_row = W_time.reshape(1, TED)
    bt_row = b_time.reshape(1, TED)
    table_pad = jnp.zeros((ZPAD, ZED), jnp.float32).at[:Z].set(table)
    ids_col = ids.reshape(N, 1)

    out = _tc_dense(cur, ids_col, time_step, w_comb, b_comb, wt_row, bt_row,
                    table_pad)
    return out.reshape(B, P, ZED + TED)


# speedup vs baseline: 95.0872x; 30.1313x over previous
import jax
import jax.numpy as jnp
from jax import lax
from jax.experimental import pallas as pl

B, P, Z, F = 1024, 50, 100, 11
ZED, TED = 64, 32
HALF = ZED // 2


def _body(zt_ref, ids_ref, ts_ref, wc_ref, tab_ref, bc_ref, wt_ref, bt_ref,
          out_ref):
    ids_p = ids_ref[...].reshape(1, B)
    zi = lax.broadcasted_iota(jnp.int32, (Z, B), 0)
    maskf = (ids_p == zi).astype(jnp.float32)
    pane = zt_ref[...].reshape(F, Z, B)
    sel = jnp.sum(pane * maskf[None], axis=1)
    zone = lax.dot_general(wc_ref[...], sel, (((1,), (0,)), ((), ())),
                           preferred_element_type=jnp.float32,
                           precision=lax.Precision.HIGHEST)
    zone = zone + lax.dot_general(tab_ref[...], maskf, (((1,), (0,)), ((), ())),
                                  preferred_element_type=jnp.float32,
                                  precision=lax.Precision.HIGHEST)
    zone = zone + bc_ref[...]
    te = wt_ref[...] * ts_ref[...] + bt_ref[...]
    out_ref[...] = jnp.concatenate([zone, te], axis=0).reshape(1, ZED + TED, B)


def kernel(zone_features, current_zone_ids, time_step, W_spatial, b_spatial,
           W_feat, b_feat, W_time, b_time, table):
    ids = current_zone_ids.astype(jnp.int32)
    zt = zone_features.transpose(1, 3, 2, 0)
    ids_t = ids.T.reshape(P, 1, B)
    ts_row = time_step.reshape(1, B)

    wc = (jnp.zeros((ZED, F), jnp.float32)
          .at[:HALF, :2].set(W_spatial)
          .at[HALF:, 2:].set(W_feat))
    tab_t = table.T
    bc = jnp.concatenate([b_spatial, b_feat]).reshape(ZED, 1)
    wt = W_time.reshape(TED, 1)
    bt = b_time.reshape(TED, 1)

    out = pl.pallas_call(
        _body,
        grid=(P,),
        in_specs=[
            pl.BlockSpec((1, F, Z, B), lambda p: (p, 0, 0, 0)),
            pl.BlockSpec((1, 1, B), lambda p: (p, 0, 0)),
            pl.BlockSpec((1, B), lambda p: (0, 0)),
            pl.BlockSpec((ZED, F), lambda p: (0, 0)),
            pl.BlockSpec((ZED, Z), lambda p: (0, 0)),
            pl.BlockSpec((ZED, 1), lambda p: (0, 0)),
            pl.BlockSpec((TED, 1), lambda p: (0, 0)),
            pl.BlockSpec((TED, 1), lambda p: (0, 0)),
        ],
        out_specs=pl.BlockSpec((1, ZED + TED, B), lambda p: (p, 0, 0)),
        out_shape=jax.ShapeDtypeStruct((P, ZED + TED, B), jnp.float32),
    )(zt, ids_t, ts_row, wc, tab_t, bc, wt, bt)

    return out.transpose(2, 0, 1)


# speedup vs baseline: 108.7857x; 1.1441x over previous
import jax
import jax.numpy as jnp
from jax import lax
from jax.experimental import pallas as pl

B, P, Z, F = 1024, 50, 100, 11
ZED, TED = 64, 32
HALF = ZED // 2


PPB = 2


def _body(zt_ref, ids_ref, ts_ref, wc_ref, tab_ref, bc_ref, wt_ref, bt_ref,
          out_ref):
    zi = lax.broadcasted_iota(jnp.int32, (Z, B), 0)
    te = wt_ref[...] * ts_ref[...] + bt_ref[...]
    for q in range(PPB):
        ids_p = ids_ref[q].reshape(1, B)
        maskf = (ids_p == zi).astype(jnp.float32)
        pane = zt_ref[q].reshape(F, Z, B)
        sel = jnp.sum(pane * maskf[None], axis=1)
        zone = lax.dot_general(wc_ref[...], sel, (((1,), (0,)), ((), ())),
                               preferred_element_type=jnp.float32,
                               precision=lax.Precision.HIGHEST)
        zone = zone + lax.dot_general(tab_ref[...], maskf,
                                      (((1,), (0,)), ((), ())),
                                      preferred_element_type=jnp.float32,
                                      precision=lax.Precision.HIGHEST)
        zone = zone + bc_ref[...]
        out_ref[q] = jnp.concatenate([zone, te], axis=0)


def kernel(zone_features, current_zone_ids, time_step, W_spatial, b_spatial,
           W_feat, b_feat, W_time, b_time, table):
    ids = current_zone_ids.astype(jnp.int32)
    zt = zone_features.transpose(1, 3, 2, 0)
    ids_t = ids.T.reshape(P, 1, B)
    ts_row = time_step.reshape(1, B)

    wc = (jnp.zeros((ZED, F), jnp.float32)
          .at[:HALF, :2].set(W_spatial)
          .at[HALF:, 2:].set(W_feat))
    tab_t = table.T
    bc = jnp.concatenate([b_spatial, b_feat]).reshape(ZED, 1)
    wt = W_time.reshape(TED, 1)
    bt = b_time.reshape(TED, 1)

    out = pl.pallas_call(
        _body,
        grid=(P // PPB,),
        in_specs=[
            pl.BlockSpec((PPB, F, Z, B), lambda p: (p, 0, 0, 0)),
            pl.BlockSpec((PPB, 1, B), lambda p: (p, 0, 0)),
            pl.BlockSpec((1, B), lambda p: (0, 0)),
            pl.BlockSpec((ZED, F), lambda p: (0, 0)),
            pl.BlockSpec((ZED, Z), lambda p: (0, 0)),
            pl.BlockSpec((ZED, 1), lambda p: (0, 0)),
            pl.BlockSpec((TED, 1), lambda p: (0, 0)),
            pl.BlockSpec((TED, 1), lambda p: (0, 0)),
        ],
        out_specs=pl.BlockSpec((PPB, ZED + TED, B), lambda p: (p, 0, 0)),
        out_shape=jax.ShapeDtypeStruct((P, ZED + TED, B), jnp.float32),
    )(zt, ids_t, ts_row, wc, tab_t, bc, wt, bt)

    return out.transpose(2, 0, 1)


# speedup vs baseline: 111.4478x; 1.0245x over previous
import jax
import jax.numpy as jnp
from jax import lax
from jax.experimental import pallas as pl

B, P, Z, F = 1024, 50, 100, 11
ZED, TED = 64, 32
HALF = ZED // 2


PPB = 5


def _body(zt_ref, ids_ref, ts_ref, wc_ref, tab_ref, bc_ref, wt_ref, bt_ref,
          out_ref):
    zi = lax.broadcasted_iota(jnp.int32, (Z, B), 0)
    te = wt_ref[...] * ts_ref[...] + bt_ref[...]
    for q in range(PPB):
        ids_p = ids_ref[q].reshape(1, B)
        maskf = (ids_p == zi).astype(jnp.float32)
        pane = zt_ref[q].reshape(F, Z, B)
        sel = jnp.sum(pane * maskf[None], axis=1)
        zone = lax.dot_general(wc_ref[...], sel, (((1,), (0,)), ((), ())),
                               preferred_element_type=jnp.float32,
                               precision=lax.Precision.HIGHEST)
        zone = zone + lax.dot_general(tab_ref[...], maskf,
                                      (((1,), (0,)), ((), ())),
                                      preferred_element_type=jnp.float32,
                                      precision=lax.Precision.HIGHEST)
        zone = zone + bc_ref[...]
        out_ref[q] = jnp.concatenate([zone, te], axis=0)


def kernel(zone_features, current_zone_ids, time_step, W_spatial, b_spatial,
           W_feat, b_feat, W_time, b_time, table):
    ids = current_zone_ids.astype(jnp.int32)
    zt = zone_features.transpose(1, 3, 2, 0)
    ids_t = ids.T.reshape(P, 1, B)
    ts_row = time_step.reshape(1, B)

    wc = (jnp.zeros((ZED, F), jnp.float32)
          .at[:HALF, :2].set(W_spatial)
          .at[HALF:, 2:].set(W_feat))
    tab_t = table.T
    bc = jnp.concatenate([b_spatial, b_feat]).reshape(ZED, 1)
    wt = W_time.reshape(TED, 1)
    bt = b_time.reshape(TED, 1)

    out = pl.pallas_call(
        _body,
        grid=(P // PPB,),
        in_specs=[
            pl.BlockSpec((PPB, F, Z, B), lambda p: (p, 0, 0, 0)),
            pl.BlockSpec((PPB, 1, B), lambda p: (p, 0, 0)),
            pl.BlockSpec((1, B), lambda p: (0, 0)),
            pl.BlockSpec((ZED, F), lambda p: (0, 0)),
            pl.BlockSpec((ZED, Z), lambda p: (0, 0)),
            pl.BlockSpec((ZED, 1), lambda p: (0, 0)),
            pl.BlockSpec((TED, 1), lambda p: (0, 0)),
            pl.BlockSpec((TED, 1), lambda p: (0, 0)),
        ],
        out_specs=pl.BlockSpec((PPB, ZED + TED, B), lambda p: (p, 0, 0)),
        out_shape=jax.ShapeDtypeStruct((P, ZED + TED, B), jnp.float32),
    )(zt, ids_t, ts_row, wc, tab_t, bc, wt, bt)

    return out.transpose(2, 0, 1)
